# baseline (speedup 1.0000x reference)
import functools

import jax
import jax.numpy as jnp
from jax import lax
from jax.experimental import pallas as pl
from jax.experimental.pallas import tpu as pltpu
from jax.experimental.pallas import tpu_sc as plsc

_B, _L, _D = 4096, 50, 128
_NC, _NS = 2, 16
_NW = _NC * _NS
_BW = _B // _NW
_NBUF = 4
_LOOK = 2
_NG = _BW // 16
_TERSE_ID = 1
_VERBOSE_ID = 2


def _sc_body(ids_hbm, table_hbm, delta3_hbm, flags_hbm, out_hbm,
             idx_v, fl_v, sel_v, drows_v, rows_v, isem, dsem, *sems):
    gsem = sems[:_NBUF]
    osem = sems[_NBUF:]
    wid = lax.axis_index("s") * _NC + lax.axis_index("c")
    base = wid * _BW

    for l in range(_L):
        pltpu.make_async_copy(
            ids_hbm.at[pl.ds(l * _B + base, _BW)], idx_v.at[l], isem).start()
    pltpu.make_async_copy(
        flags_hbm.at[pl.ds(wid * _L * 16, _L * 16)], fl_v, isem).start()
    for l in range(_L):
        pltpu.make_async_copy(
            ids_hbm.at[pl.ds(base, _BW)], idx_v.at[l], isem).wait()
    pltpu.make_async_copy(
        flags_hbm.at[pl.ds(0, _L * 16)], fl_v, isem).wait()

    def gdesc(c, b):
        return pltpu.make_async_copy(
            table_hbm.at[idx_v.at[c]], rows_v.at[b], gsem[b])

    def odesc(c, b):
        return pltpu.make_async_copy(
            rows_v.at[b], out_hbm.at[c, pl.ds(base, _BW)], osem[b])

    def process(c, b):
        gdesc(c, b).wait()
        match = fl_v[pl.ds(16 * c, 16)][0]

        @pl.when(match > 0)
        def _apply_delta():
            for g in range(_NG):
                v = idx_v[c, pl.ds(16 * g, 16)]
                s = (jnp.where(v == _TERSE_ID, 1, 0)
                     + jnp.where(v == _VERBOSE_ID, 2, 0)).astype(jnp.int32)
                sel_v[pl.ds(16 * g, 16)] = s
            pltpu.make_async_copy(delta3_hbm.at[sel_v], drows_v, dsem).start()
            pltpu.make_async_copy(delta3_hbm.at[sel_v], drows_v, dsem).wait()

            def add_row(r, car):
                for j in range(_D // 16):
                    sl = pl.ds(16 * j, 16)
                    rows_v[b, r, sl] = rows_v[b, r, sl] + drows_v[r, sl]
                return car

            lax.fori_loop(0, _BW, add_row, 0)

        odesc(c, b).start()

    for c in range(_LOOK):
        gdesc(c, c).start()
    for c in range(_LOOK):
        gdesc(c + _LOOK, c + _LOOK).start()
        process(c, c)

    n_steady = _L - 2 * _LOOK - ((_L - 2 * _LOOK) % _NBUF)
    def group(c0, carry):
        for k in range(_NBUF):
            b = (_LOOK + k) % _NBUF
            bn = (_LOOK + k + _LOOK) % _NBUF
            c = c0 + k
            odesc(c - _LOOK, bn).wait()
            gdesc(c + _LOOK, bn).start()
            process(c, b)
        return carry

    lax.fori_loop(0, n_steady // _NBUF,
                  lambda i, car: group(_LOOK + i * _NBUF, car), 0)

    for c in range(_LOOK + n_steady, _L - _LOOK):
        b, bn = c % _NBUF, (c + _LOOK) % _NBUF
        odesc(c + _LOOK - _NBUF, bn).wait()
        gdesc(c + _LOOK, bn).start()
        process(c, b)
    for c in range(_L - _LOOK, _L):
        process(c, c % _NBUF)
    for b in range(_NBUF):
        odesc(0, b).wait()


def kernel(input_ids, base_table, style_delta):
    ids_t = input_ids.astype(jnp.int32).T
    ids = ids_t.reshape(-1)
    delta3 = jnp.concatenate(
        [jnp.zeros((1, _D), style_delta.dtype), style_delta], axis=0)
    is_style = (ids_t == _TERSE_ID) | (ids_t == _VERBOSE_ID)
    stripe_flag = jnp.any(is_style.reshape(_L, _NW, _BW), axis=-1).T
    flags = jnp.broadcast_to(
        stripe_flag.astype(jnp.int32)[:, :, None], (_NW, _L, 16)).reshape(-1)

    mesh = plsc.VectorSubcoreMesh(core_axis_name="c", subcore_axis_name="s")
    run = functools.partial(
        pl.kernel,
        mesh=mesh,
        out_type=jax.ShapeDtypeStruct((_L, _B, _D), jnp.float32),
        compiler_params=pltpu.CompilerParams(use_tc_tiling_on_sc=True),
        scratch_types=[
            pltpu.VMEM((_L, _BW), jnp.int32),
            pltpu.VMEM((_L * 16,), jnp.int32),
            pltpu.VMEM((_BW,), jnp.int32),
            pltpu.VMEM((_BW, _D), jnp.float32),
            pltpu.VMEM((_NBUF, _BW, _D), jnp.float32),
            pltpu.SemaphoreType.DMA,
            pltpu.SemaphoreType.DMA,
        ] + [pltpu.SemaphoreType.DMA] * (2 * _NBUF),
    )(_sc_body)
    return run(ids, base_table, delta3, flags).transpose(1, 0, 2)


# speedup vs baseline: 1.1368x; 1.1368x over previous
import functools

import jax
import jax.numpy as jnp
from jax import lax
from jax.experimental import pallas as pl
from jax.experimental.pallas import tpu as pltpu
from jax.experimental.pallas import tpu_sc as plsc

_B, _L, _D = 4096, 50, 128
_NC, _NS = 2, 16
_NW = _NC * _NS
_BW = _B // _NW
_NBUF = 4
_LOOK = 2
_NG = _BW // 16
_FW = (_L + 1) * 16
_TERSE_ID = 1
_VERBOSE_ID = 2


def _sc_body(ids_hbm, table_hbm, delta3_hbm, flags_hbm, out_hbm,
             idx_v, fl_v, sel_v, drows_v, rows_v, isem, dsem, *sems):
    gsem = sems[:_NBUF]
    osem = sems[_NBUF:]
    wid = lax.axis_index("s") * _NC + lax.axis_index("c")
    base = wid * _BW

    for l in range(_L):
        pltpu.make_async_copy(
            ids_hbm.at[pl.ds(l * _B + base, _BW)], idx_v.at[l], isem).start()
    pltpu.make_async_copy(
        flags_hbm.at[pl.ds(wid * _FW, _FW)], fl_v, isem).start()
    for l in range(_L):
        pltpu.make_async_copy(
            ids_hbm.at[pl.ds(base, _BW)], idx_v.at[l], isem).wait()
    pltpu.make_async_copy(
        flags_hbm.at[pl.ds(0, _FW)], fl_v, isem).wait()

    def gdesc(c, b):
        return pltpu.make_async_copy(
            table_hbm.at[idx_v.at[c]], rows_v.at[b], gsem[b])

    def odesc(c, b):
        return pltpu.make_async_copy(
            rows_v.at[b], out_hbm.at[c, pl.ds(base, _BW)], osem[b])

    for c in range(_LOOK):
        gdesc(c, c).start()
    for c in range(_LOOK):
        gdesc(c + _LOOK, c + _LOOK).start()
        gdesc(c, c).wait()
        odesc(c, c).start()

    n_steady = _L - 2 * _LOOK - ((_L - 2 * _LOOK) % _NBUF)
    def group(c0, carry):
        for k in range(_NBUF):
            b = (_LOOK + k) % _NBUF
            bn = (_LOOK + k + _LOOK) % _NBUF
            c = c0 + k
            odesc(c - _LOOK, bn).wait()
            gdesc(c + _LOOK, bn).start()
            gdesc(c, b).wait()
            odesc(c, b).start()
        return carry

    lax.fori_loop(0, n_steady // _NBUF,
                  lambda i, car: group(_LOOK + i * _NBUF, car), 0)

    for c in range(_LOOK + n_steady, _L - _LOOK):
        b, bn = c % _NBUF, (c + _LOOK) % _NBUF
        odesc(c + _LOOK - _NBUF, bn).wait()
        gdesc(c + _LOOK, bn).start()
        gdesc(c, b).wait()
        odesc(c, b).start()
    for c in range(_L - _LOOK, _L):
        b = c % _NBUF
        gdesc(c, b).wait()
        odesc(c, b).start()
    for b in range(_NBUF):
        odesc(0, b).wait()

    wmatch = fl_v[pl.ds(0, 16)][0]

    @pl.when(wmatch > 0)
    def _fix_planes():
        def fix_plane(c, car):
            m = fl_v[pl.ds(16 + 16 * c, 16)][0]

            @pl.when(m > 0)
            def _apply_delta():
                pltpu.sync_copy(out_hbm.at[c, pl.ds(base, _BW)], rows_v.at[0])
                for g in range(_NG):
                    v = idx_v[c, pl.ds(16 * g, 16)]
                    s = (jnp.where(v == _TERSE_ID, 1, 0)
                         + jnp.where(v == _VERBOSE_ID, 2, 0)).astype(jnp.int32)
                    sel_v[pl.ds(16 * g, 16)] = s
                cp = pltpu.make_async_copy(delta3_hbm.at[sel_v], drows_v, dsem)
                cp.start()
                cp.wait()

                def add_row(r, car2):
                    for j in range(_D // 16):
                        sl = pl.ds(16 * j, 16)
                        rows_v[0, r, sl] = rows_v[0, r, sl] + drows_v[r, sl]
                    return car2

                lax.fori_loop(0, _BW, add_row, 0)
                pltpu.sync_copy(rows_v.at[0], out_hbm.at[c, pl.ds(base, _BW)])

            return car

        lax.fori_loop(0, _L, fix_plane, 0)


def kernel(input_ids, base_table, style_delta):
    ids_t = input_ids.astype(jnp.int32).T
    ids = ids_t.reshape(-1)
    delta3 = jnp.concatenate(
        [jnp.zeros((1, _D), style_delta.dtype), style_delta], axis=0)
    is_style = (ids_t == _TERSE_ID) | (ids_t == _VERBOSE_ID)
    plane_flag = jnp.any(is_style.reshape(_L, _NW, _BW), axis=-1).T
    worker_flag = jnp.any(plane_flag, axis=-1)
    allf = jnp.concatenate(
        [worker_flag[:, None], plane_flag], axis=1).astype(jnp.int32)
    flags = jnp.broadcast_to(allf[:, :, None], (_NW, _L + 1, 16)).reshape(-1)

    mesh = plsc.VectorSubcoreMesh(core_axis_name="c", subcore_axis_name="s")
    run = functools.partial(
        pl.kernel,
        mesh=mesh,
        out_type=jax.ShapeDtypeStruct((_L, _B, _D), jnp.float32),
        compiler_params=pltpu.CompilerParams(use_tc_tiling_on_sc=True),
        scratch_types=[
            pltpu.VMEM((_L, _BW), jnp.int32),
            pltpu.VMEM((_FW,), jnp.int32),
            pltpu.VMEM((_BW,), jnp.int32),
            pltpu.VMEM((_BW, _D), jnp.float32),
            pltpu.VMEM((_NBUF, _BW, _D), jnp.float32),
            pltpu.SemaphoreType.DMA,
            pltpu.SemaphoreType.DMA,
        ] + [pltpu.SemaphoreType.DMA] * (2 * _NBUF),
    )(_sc_body)
    return run(ids, base_table, delta3, flags).transpose(1, 0, 2)
